# baseline (speedup 1.0000x reference)
import functools

import jax
import jax.numpy as jnp
from jax import lax
from jax.experimental import pallas as pl
from jax.experimental.pallas import tpu as pltpu
from jax.experimental.pallas import tpu_sc as plsc

_NC, _NS = 2, 16
_NT = _NC * _NS
_K = 8
_C = 128
_BM = 256


def _mesh():
    return plsc.VectorSubcoreMesh(core_axis_name="c", subcore_axis_name="s")


def _part_call(srcF, dstF, npad, ch):
    half = npad // 2
    trash = half + 512
    ept = ch * _C
    nv = ept // 16

    @functools.partial(
        pl.kernel,
        out_type=[jax.ShapeDtypeStruct((2, _NT, ept), jnp.int32),
                  jax.ShapeDtypeStruct((2, _NT, ept), jnp.int32),
                  jax.ShapeDtypeStruct((2, _NT, 16), jnp.int32)],
        mesh=_mesh(),
        compiler_params=pltpu.CompilerParams(needs_layout_passes=False),
        scratch_types=[
            pltpu.VMEM((ept,), jnp.int32),
            pltpu.VMEM((ept,), jnp.int32),
            pltpu.VMEM((ept + 16,), jnp.int32),
            pltpu.VMEM((ept + 16,), jnp.int32),
            pltpu.VMEM((ept + 16,), jnp.int32),
            pltpu.VMEM((ept + 16,), jnp.int32),
            pltpu.VMEM((16,), jnp.int32),
        ],
    )
    def k(src_hbm, dst_hbm, srcL_hbm, dstL_hbm, cnt_hbm,
          siv, div, oAs, oAd, oBs, oBd, cv):
        co = lax.axis_index("c")
        s = lax.axis_index("s")
        blk = co * _NS + s
        pltpu.sync_copy(src_hbm.at[blk], siv)
        pltpu.sync_copy(dst_hbm.at[blk], div)

        zer = jnp.zeros((16,), jnp.int32)
        tra = jnp.full((16,), trash, jnp.int32)

        @pl.loop(0, ept + 16, step=16)
        def _pre(i):
            oAs[pl.ds(i, 16)] = zer
            oAd[pl.ds(i, 16)] = tra
            oBs[pl.ds(i, 16)] = zer
            oBd[pl.ds(i, 16)] = tra

        def body(j, carry):
            nA, nB = carry
            d = div[pl.ds(j * 16, 16)]
            sv = siv[pl.ds(j * 16, 16)]
            mA = d < half
            mB = jnp.logical_not(mA)
            plsc.store_compressed(oAs.at[pl.ds(nA, 16)], sv, mask=mA)
            plsc.store_compressed(oAd.at[pl.ds(nA, 16)], d, mask=mA)
            plsc.store_compressed(oBs.at[pl.ds(nB, 16)], sv, mask=mB)
            plsc.store_compressed(oBd.at[pl.ds(nB, 16)], d - half, mask=mB)
            nA = nA + jnp.max(plsc.all_reduce_population_count(mA))
            nB = nB + jnp.max(plsc.all_reduce_population_count(mB))
            return nA, nB

        nA, nB = lax.fori_loop(0, nv, body, (jnp.int32(0), jnp.int32(0)))

        i16 = lax.iota(jnp.int32, 16)
        cv[...] = jnp.where(i16 == 0, nA, 0)
        pltpu.sync_copy(cv, cnt_hbm.at[0, blk])
        cv[...] = jnp.where(i16 == 0, nB, 0)
        pltpu.sync_copy(cv, cnt_hbm.at[1, blk])
        pltpu.sync_copy(oAs.at[pl.ds(0, ept)], srcL_hbm.at[0, blk])
        pltpu.sync_copy(oAd.at[pl.ds(0, ept)], dstL_hbm.at[0, blk])
        pltpu.sync_copy(oBs.at[pl.ds(0, ept)], srcL_hbm.at[1, blk])
        pltpu.sync_copy(oBd.at[pl.ds(0, ept)], dstL_hbm.at[1, blk])

    return k(srcF, dstF)


def _cons_call(srcL, dstL, cnt, npad, ch):
    bw = npad // _NT
    ept = ch * _C
    nchm = _NT * ch

    @functools.partial(
        pl.kernel,
        out_type=[jax.ShapeDtypeStruct((_NT, nchm, _C), jnp.int32),
                  jax.ShapeDtypeStruct((_NT, nchm, _C), jnp.int32),
                  jax.ShapeDtypeStruct((_NT, 16), jnp.int32)],
        mesh=_mesh(),
        compiler_params=pltpu.CompilerParams(needs_layout_passes=False),
        scratch_types=[
            pltpu.VMEM((ept,), jnp.int32),
            pltpu.VMEM((ept,), jnp.int32),
            pltpu.VMEM((ept + 144,), jnp.int32),
            pltpu.VMEM((ept + 144,), jnp.int32),
            pltpu.VMEM((_C,), jnp.int32),
            pltpu.VMEM((_C,), jnp.int32),
            pltpu.VMEM((16,), jnp.int32),
        ],
    )
    def k(srcL_hbm, dstL_hbm, cnt_hbm, srcC_hbm, dstC_hbm, len_hbm,
          siv, div, oS, oD, tS, tD, cv):
        co = lax.axis_index("c")
        s = lax.axis_index("s")
        blk = co * _NS + s
        hf = blk // _NS
        lo = (blk % _NS) * bw

        zer = jnp.zeros((16,), jnp.int32)
        tra = jnp.full((16,), bw, jnp.int32)

        @pl.loop(0, _C, step=16)
        def _pret(i):
            tS[pl.ds(i, 16)] = zer
            tD[pl.ds(i, 16)] = tra

        @pl.loop(0, ept + 144, step=16)
        def _pres(i):
            oS[pl.ds(i, 16)] = zer

        cur = jnp.int32(0)
        for p in range(_NT):
            pltpu.sync_copy(srcL_hbm.at[hf, p], siv)
            pltpu.sync_copy(dstL_hbm.at[hf, p], div)
            pltpu.sync_copy(cnt_hbm.at[hf, p], cv)
            c_p = cv[...][0]

            @pl.loop(0, ept + 144, step=16)
            def _pred(i):
                oD[pl.ds(i, 16)] = tra

            def body(g, n):
                d = div[pl.ds(g * 16, 16)]
                sv = siv[pl.ds(g * 16, 16)]
                m = (d >= lo) & (d < lo + bw)
                plsc.store_compressed(oS.at[pl.ds(n, 16)], sv, mask=m)
                plsc.store_compressed(oD.at[pl.ds(n, 16)], d - lo, mask=m)
                return n + jnp.max(plsc.all_reduce_population_count(m))

            nk = lax.fori_loop(0, (c_p + 15) // 16, body, jnp.int32(0))
            nfl = (nk + _C - 1) // _C

            def fl(q, carry):
                pltpu.sync_copy(oS.at[pl.ds(q * _C, _C)],
                                srcC_hbm.at[blk, cur + q])
                pltpu.sync_copy(oD.at[pl.ds(q * _C, _C)],
                                dstC_hbm.at[blk, cur + q])
                return carry

            lax.fori_loop(0, nfl, fl, jnp.int32(0))
            cur = cur + nfl

        @pl.when(lax.rem(cur, 2) == 1)
        def _padchunk():
            pltpu.sync_copy(tS, srcC_hbm.at[blk, cur])
            pltpu.sync_copy(tD, dstC_hbm.at[blk, cur])

        cur = cur + lax.rem(cur, 2)
        i16 = lax.iota(jnp.int32, 16)
        cv[...] = jnp.where(i16 == 0, cur * _C, 0)
        pltpu.sync_copy(cv, len_hbm.at[blk])

    return k(srcL, dstL, cnt)


def _sc_call(mode, hn, srcC, dstC, lenC, npad, ch):
    bw = npad // _NT
    accn = bw + 8

    @functools.partial(
        pl.kernel,
        out_type=jax.ShapeDtypeStruct((npad, 128), jnp.float32),
        mesh=_mesh(),
        scratch_types=[
            pltpu.VMEM((_C,), jnp.int32),
            pltpu.VMEM((_C,), jnp.int32),
            pltpu.VMEM((_C,), jnp.int32),
            pltpu.VMEM((_C,), jnp.int32),
            pltpu.VMEM((_C, 128), jnp.float32),
            pltpu.VMEM((_C, 128), jnp.float32),
            pltpu.VMEM((accn, 128), jnp.float32),
            pltpu.VMEM((16,), jnp.int32),
            pltpu.SemaphoreType.DMA,
            pltpu.SemaphoreType.DMA,
        ],
    )
    def k(hn_hbm, srcC_hbm, dstC_hbm, len_hbm, out_hbm,
          si0, si1, di0, di1, rows0, rows1, acc, lv, g0, g1):
        co = lax.axis_index("c")
        s = lax.axis_index("s")
        blk = co * _NS + s

        @pl.loop(0, accn)
        def _z(i):
            @pl.loop(0, 128, step=16)
            def _z2(q):
                acc[i, pl.ds(q, 16)] = jnp.zeros((16,), jnp.float32)

        pltpu.sync_copy(len_hbm.at[blk], lv)
        nch = lv[...][0] // _C
        nit = nch // 2

        one = jnp.ones((16,), jnp.float32)

        if mode == "deg":
            def dbody(i, carry):
                pltpu.sync_copy(dstC_hbm.at[blk, i], di0)

                @pl.loop(0, 8)
                def _g(g):
                    dv = di0[pl.ds(g * 16, 16)]
                    for kk in range(16):
                        plsc.addupdate(acc.at[dv[kk], pl.ds(0, 16)], one)

                return carry

            lax.fori_loop(0, nch, dbody, jnp.int32(0))
        else:
            def accum(rows, di):
                @pl.loop(0, 8)
                def _g(g):
                    dv = di[pl.ds(g * 16, 16)]
                    for kk in range(16):
                        r = g * 16 + kk
                        dl = dv[kk]
                        for m in range(8):
                            plsc.addupdate(acc.at[dl, pl.ds(m * 16, 16)],
                                           rows[r, pl.ds(m * 16, 16)])

            @pl.when(nch > 0)
            def _prop():
                pltpu.sync_copy(srcC_hbm.at[blk, 0], si0)
                pltpu.sync_copy(dstC_hbm.at[blk, 0], di0)
                pltpu.async_copy(hn_hbm.at[si0], rows0, g0)
                c1 = lax.rem(jnp.int32(1), nch)
                pltpu.sync_copy(srcC_hbm.at[blk, c1], si1)
                pltpu.sync_copy(dstC_hbm.at[blk, c1], di1)
                pltpu.async_copy(hn_hbm.at[si1], rows1, g1)

                def pbody(i, carry):
                    j = 2 * i
                    pltpu.make_async_copy(hn_hbm.at[si0], rows0, g0).wait()
                    accum(rows0, di0)
                    jn = lax.rem(j + 2, nch)
                    pltpu.sync_copy(srcC_hbm.at[blk, jn], si0)
                    pltpu.sync_copy(dstC_hbm.at[blk, jn], di0)
                    pltpu.async_copy(hn_hbm.at[si0], rows0, g0)

                    pltpu.make_async_copy(hn_hbm.at[si1], rows1, g1).wait()
                    accum(rows1, di1)
                    jn1 = lax.rem(j + 3, nch)
                    pltpu.sync_copy(srcC_hbm.at[blk, jn1], si1)
                    pltpu.sync_copy(dstC_hbm.at[blk, jn1], di1)
                    pltpu.async_copy(hn_hbm.at[si1], rows1, g1)
                    return carry

                lax.fori_loop(0, nit, pbody, jnp.int32(0))

                pltpu.make_async_copy(hn_hbm.at[si0], rows0, g0).wait()
                pltpu.make_async_copy(hn_hbm.at[si1], rows1, g1).wait()

        pltpu.sync_copy(acc.at[pl.ds(0, bw)],
                        out_hbm.at[pl.ds(blk * bw, bw)])

    return k(hn, srcC, dstC, lenC)


def _mm_body(x_ref, w_ref, b_ref, dp_ref, h_ref, hn_ref, dv_ref):
    h = jnp.dot(x_ref[...], w_ref[...],
                preferred_element_type=jnp.float32) + b_ref[...]
    deg = dp_ref[:, :1] + 1.0
    dv = jnp.broadcast_to(lax.rsqrt(deg), h.shape)
    h_ref[...] = h
    hn_ref[...] = h * dv
    dv_ref[...] = dv


def _mm_call(xp, w, b2, degp, npad):
    f = jax.ShapeDtypeStruct((npad, 128), jnp.float32)
    return pl.pallas_call(
        _mm_body,
        grid=(npad // _BM,),
        in_specs=[
            pl.BlockSpec((_BM, 128), lambda i: (i, 0)),
            pl.BlockSpec((128, 128), lambda i: (0, 0)),
            pl.BlockSpec((1, 128), lambda i: (0, 0)),
            pl.BlockSpec((_BM, 128), lambda i: (i, 0)),
        ],
        out_specs=[pl.BlockSpec((_BM, 128), lambda i: (i, 0))] * 3,
        out_shape=[f, f, f],
    )(xp, w, b2, degp)


def _comb_body(cl, dl, last, p_ref, hn_ref, h0_ref, dv_ref, h_ref, hno_ref):
    dv = dv_ref[...]
    agg = p_ref[...] + hn_ref[...]
    h = cl * (agg * dv) + dl * h0_ref[...]
    h_ref[...] = h
    if last:
        hno_ref[...] = h
    else:
        hno_ref[...] = h * dv


def _comb_call(p, hn, h0, dvb, cl, dl, last, npad):
    f = jax.ShapeDtypeStruct((npad, 128), jnp.float32)
    return pl.pallas_call(
        functools.partial(_comb_body, cl, dl, last),
        grid=(npad // _BM,),
        in_specs=[pl.BlockSpec((_BM, 128), lambda i: (i, 0))] * 4,
        out_specs=[pl.BlockSpec((_BM, 128), lambda i: (i, 0))] * 2,
        out_shape=[f, f],
    )(p, hn, h0, dvb)


def kernel(features, edge_index, W, b):
    n, d = features.shape
    e = edge_index.shape[1]
    npad = (n // 2560 + 1) * 2560

    ept0 = -(-e // _NT)
    ch = -(-ept0 // _C)
    ch += ch % 2
    ept = ch * _C

    srcF = jnp.pad(edge_index[0], (0, _NT * ept - e)).reshape(_NT, ept)
    dstF = jnp.pad(edge_index[1], (0, _NT * ept - e),
                   constant_values=npad).reshape(_NT, ept)
    xp = jnp.pad(features, ((0, npad - n), (0, 0)))
    b2 = b.reshape(1, d)

    srcL, dstL, cnt = _part_call(srcF, dstF, npad, ch)
    srcC, dstC, lenC = _cons_call(srcL, dstL, cnt, npad, ch)

    degp = _sc_call("deg", xp, srcC, dstC, lenC, npad, ch)
    h0, hn, dvb = _mm_call(xp, W, b2, degp, npad)

    h = h0
    for l in range(1, _K + 1):
        p = _sc_call("prop", hn, srcC, dstC, lenC, npad, ch)
        h, hn = _comb_call(p, hn, h0, dvb,
                           l / (l + 1.0), 1.0 / (l + 1.0), l == _K, npad)
    return h[:n]


# speedup vs baseline: 1.5946x; 1.5946x over previous
import functools

import jax
import jax.numpy as jnp
from jax import lax
from jax.experimental import pallas as pl
from jax.experimental.pallas import tpu as pltpu
from jax.experimental.pallas import tpu_sc as plsc

_NC, _NS = 2, 16
_NT = _NC * _NS
_K = 8
_C = 128
_BM = 256


def _mesh():
    return plsc.VectorSubcoreMesh(core_axis_name="c", subcore_axis_name="s")


def _part_call(srcF, dstF, npad, ch):
    half = npad // 2
    trash = half + 512
    ept = ch * _C
    nv = ept // 16

    @functools.partial(
        pl.kernel,
        out_type=[jax.ShapeDtypeStruct((2, _NT, ept), jnp.int32),
                  jax.ShapeDtypeStruct((2, _NT, ept), jnp.int32),
                  jax.ShapeDtypeStruct((2, _NT, 16), jnp.int32)],
        mesh=_mesh(),
        compiler_params=pltpu.CompilerParams(needs_layout_passes=False),
        scratch_types=[
            pltpu.VMEM((ept,), jnp.int32),
            pltpu.VMEM((ept,), jnp.int32),
            pltpu.VMEM((ept + 16,), jnp.int32),
            pltpu.VMEM((ept + 16,), jnp.int32),
            pltpu.VMEM((ept + 16,), jnp.int32),
            pltpu.VMEM((ept + 16,), jnp.int32),
            pltpu.VMEM((16,), jnp.int32),
        ],
    )
    def k(src_hbm, dst_hbm, srcL_hbm, dstL_hbm, cnt_hbm,
          siv, div, oAs, oAd, oBs, oBd, cv):
        co = lax.axis_index("c")
        s = lax.axis_index("s")
        blk = co * _NS + s
        pltpu.sync_copy(src_hbm.at[blk], siv)
        pltpu.sync_copy(dst_hbm.at[blk], div)

        zer = jnp.zeros((16,), jnp.int32)
        tra = jnp.full((16,), trash, jnp.int32)

        @pl.loop(0, ept + 16, step=16)
        def _pre(i):
            oAs[pl.ds(i, 16)] = zer
            oAd[pl.ds(i, 16)] = tra
            oBs[pl.ds(i, 16)] = zer
            oBd[pl.ds(i, 16)] = tra

        def body(j, carry):
            nA, nB = carry
            d = div[pl.ds(j * 16, 16)]
            sv = siv[pl.ds(j * 16, 16)]
            mA = d < half
            mB = jnp.logical_not(mA)
            plsc.store_compressed(oAs.at[pl.ds(nA, 16)], sv, mask=mA)
            plsc.store_compressed(oAd.at[pl.ds(nA, 16)], d, mask=mA)
            plsc.store_compressed(oBs.at[pl.ds(nB, 16)], sv, mask=mB)
            plsc.store_compressed(oBd.at[pl.ds(nB, 16)], d - half, mask=mB)
            nA = nA + jnp.max(plsc.all_reduce_population_count(mA))
            nB = nB + jnp.max(plsc.all_reduce_population_count(mB))
            return nA, nB

        nA, nB = lax.fori_loop(0, nv, body, (jnp.int32(0), jnp.int32(0)))

        i16 = lax.iota(jnp.int32, 16)
        cv[...] = jnp.where(i16 == 0, nA, 0)
        pltpu.sync_copy(cv, cnt_hbm.at[0, blk])
        cv[...] = jnp.where(i16 == 0, nB, 0)
        pltpu.sync_copy(cv, cnt_hbm.at[1, blk])
        pltpu.sync_copy(oAs.at[pl.ds(0, ept)], srcL_hbm.at[0, blk])
        pltpu.sync_copy(oAd.at[pl.ds(0, ept)], dstL_hbm.at[0, blk])
        pltpu.sync_copy(oBs.at[pl.ds(0, ept)], srcL_hbm.at[1, blk])
        pltpu.sync_copy(oBd.at[pl.ds(0, ept)], dstL_hbm.at[1, blk])

    return k(srcF, dstF)


def _sc_call(mode, hn, srcC, dstC, cnt, npad, ch):
    half = npad // 2
    accn = half + 640
    bw = half // _NS

    @functools.partial(
        pl.kernel,
        out_type=jax.ShapeDtypeStruct((npad, 128), jnp.float32),
        mesh=_mesh(),
        scratch_types=[
            pltpu.VMEM((_C,), jnp.int32),
            pltpu.VMEM((_C,), jnp.int32),
            pltpu.VMEM((_C,), jnp.int32),
            pltpu.VMEM((_C,), jnp.int32),
            pltpu.VMEM((_C, 128), jnp.float32),
            pltpu.VMEM((_C, 128), jnp.float32),
            pltpu.VMEM_SHARED((accn, 128), jnp.float32),
            pltpu.VMEM((16,), jnp.int32),
            pltpu.SemaphoreType.DMA,
            pltpu.SemaphoreType.DMA,
        ],
    )
    def k(hn_hbm, srcC_hbm, dstC_hbm, cnt_hbm, out_hbm,
          si0, si1, di0, di1, rows0, rows1, acc, cv, g0, g1):
        co = lax.axis_index("c")
        s = lax.axis_index("s")

        zrow = accn // _NS
        z16 = jnp.zeros((16,), jnp.float32)

        @pl.loop(0, _C)
        def _z(i):
            @pl.loop(0, 128, step=16)
            def _z2(q):
                rows0[i, pl.ds(q, 16)] = z16

        off = 0
        while off < zrow:
            n = min(_C, zrow - off)
            pltpu.sync_copy(rows0.at[pl.ds(0, n)],
                            acc.at[pl.ds(s * zrow + off, n)])
            off += n

        plsc.subcore_barrier()

        one = jnp.ones((16,), jnp.float32)
        if mode == "deg":
            @pl.loop(0, _C)
            def _ones(i):
                @pl.loop(0, 128, step=16)
                def _ones2(q):
                    rows0[i, pl.ds(q, 16)] = one

        for li in range(2):
            p = s + li * _NS
            pltpu.sync_copy(cnt_hbm.at[co, p], cv)
            c_p = cv[...][0]
            nch = (c_p + _C - 1) // _C

            if mode == "deg":
                def dbody(i, carry):
                    pltpu.sync_copy(dstC_hbm.at[co, p, i], di0)
                    pltpu.sync_copy(rows0, acc.at[di0], add=True)
                    return carry

                lax.fori_loop(0, nch, dbody, jnp.int32(0))
            else:
                nch = nch + lax.rem(nch, 2)

                @pl.when(nch > 0)
                def _prop():
                    pltpu.sync_copy(srcC_hbm.at[co, p, 0], si0)
                    pltpu.sync_copy(dstC_hbm.at[co, p, 0], di0)
                    pltpu.async_copy(hn_hbm.at[si0], rows0, g0)
                    c1 = lax.rem(jnp.int32(1), nch)
                    pltpu.sync_copy(srcC_hbm.at[co, p, c1], si1)
                    pltpu.sync_copy(dstC_hbm.at[co, p, c1], di1)
                    pltpu.async_copy(hn_hbm.at[si1], rows1, g1)

                    def pbody(i, carry):
                        j = 2 * i
                        pltpu.make_async_copy(hn_hbm.at[si0], rows0, g0).wait()
                        pltpu.sync_copy(rows0, acc.at[di0], add=True)
                        jn = lax.rem(j + 2, nch)
                        pltpu.sync_copy(srcC_hbm.at[co, p, jn], si0)
                        pltpu.sync_copy(dstC_hbm.at[co, p, jn], di0)
                        pltpu.async_copy(hn_hbm.at[si0], rows0, g0)

                        pltpu.make_async_copy(hn_hbm.at[si1], rows1, g1).wait()
                        pltpu.sync_copy(rows1, acc.at[di1], add=True)
                        jn1 = lax.rem(j + 3, nch)
                        pltpu.sync_copy(srcC_hbm.at[co, p, jn1], si1)
                        pltpu.sync_copy(dstC_hbm.at[co, p, jn1], di1)
                        pltpu.async_copy(hn_hbm.at[si1], rows1, g1)
                        return carry

                    lax.fori_loop(0, nch // 2, pbody, jnp.int32(0))

                    pltpu.make_async_copy(hn_hbm.at[si0], rows0, g0).wait()
                    pltpu.make_async_copy(hn_hbm.at[si1], rows1, g1).wait()

        plsc.subcore_barrier()
        pltpu.sync_copy(acc.at[pl.ds(s * bw, bw)],
                        out_hbm.at[pl.ds(co * half + s * bw, bw)])

    return k(hn, srcC, dstC, cnt)


def _mm_body(x_ref, w_ref, b_ref, dp_ref, h_ref, hn_ref, dv_ref):
    h = jnp.dot(x_ref[...], w_ref[...],
                preferred_element_type=jnp.float32) + b_ref[...]
    deg = dp_ref[:, :1] + 1.0
    dv = jnp.broadcast_to(lax.rsqrt(deg), h.shape)
    h_ref[...] = h
    hn_ref[...] = h * dv
    dv_ref[...] = dv


def _mm_call(xp, w, b2, degp, npad):
    f = jax.ShapeDtypeStruct((npad, 128), jnp.float32)
    return pl.pallas_call(
        _mm_body,
        grid=(npad // _BM,),
        in_specs=[
            pl.BlockSpec((_BM, 128), lambda i: (i, 0)),
            pl.BlockSpec((128, 128), lambda i: (0, 0)),
            pl.BlockSpec((1, 128), lambda i: (0, 0)),
            pl.BlockSpec((_BM, 128), lambda i: (i, 0)),
        ],
        out_specs=[pl.BlockSpec((_BM, 128), lambda i: (i, 0))] * 3,
        out_shape=[f, f, f],
    )(xp, w, b2, degp)


def _comb_body(cl, dl, last, p_ref, hn_ref, h0_ref, dv_ref, h_ref, hno_ref):
    dv = dv_ref[...]
    agg = p_ref[...] + hn_ref[...]
    h = cl * (agg * dv) + dl * h0_ref[...]
    h_ref[...] = h
    if last:
        hno_ref[...] = h
    else:
        hno_ref[...] = h * dv


def _comb_call(p, hn, h0, dvb, cl, dl, last, npad):
    f = jax.ShapeDtypeStruct((npad, 128), jnp.float32)
    return pl.pallas_call(
        functools.partial(_comb_body, cl, dl, last),
        grid=(npad // _BM,),
        in_specs=[pl.BlockSpec((_BM, 128), lambda i: (i, 0))] * 4,
        out_specs=[pl.BlockSpec((_BM, 128), lambda i: (i, 0))] * 2,
        out_shape=[f, f],
    )(p, hn, h0, dvb)


def kernel(features, edge_index, W, b):
    n, d = features.shape
    e = edge_index.shape[1]
    npad = (n // 2560 + 1) * 2560

    ept0 = -(-e // _NT)
    ch = -(-ept0 // _C)
    ch += ch % 2
    ept = ch * _C

    srcF = jnp.pad(edge_index[0], (0, _NT * ept - e)).reshape(_NT, ept)
    dstF = jnp.pad(edge_index[1], (0, _NT * ept - e),
                   constant_values=npad).reshape(_NT, ept)
    xp = jnp.pad(features, ((0, npad - n), (0, 0)))
    b2 = b.reshape(1, d)

    srcL, dstL, cnt = _part_call(srcF, dstF, npad, ch)
    srcC = srcL.reshape(2, _NT, ept // _C, _C)
    dstC = dstL.reshape(2, _NT, ept // _C, _C)

    degp = _sc_call("deg", xp, srcC, dstC, cnt, npad, ch)
    h0, hn, dvb = _mm_call(xp, W, b2, degp, npad)

    h = h0
    for l in range(1, _K + 1):
        p = _sc_call("prop", hn, srcC, dstC, cnt, npad, ch)
        h, hn = _comb_call(p, hn, h0, dvb,
                           l / (l + 1.0), 1.0 / (l + 1.0), l == _K, npad)
    return h[:n]


# speedup vs baseline: 1.6487x; 1.0339x over previous
import functools

import jax
import jax.numpy as jnp
from jax import lax
from jax.experimental import pallas as pl
from jax.experimental.pallas import tpu as pltpu
from jax.experimental.pallas import tpu_sc as plsc

_NC, _NS = 2, 16
_NT = _NC * _NS
_K = 8
_C = 128
_BM = 256


def _mesh():
    return plsc.VectorSubcoreMesh(core_axis_name="c", subcore_axis_name="s")


def _part_call(srcF, dstF, npad, ch):
    half = npad // 2
    trash = half + 512
    ept = ch * _C
    nv = ept // 16

    @functools.partial(
        pl.kernel,
        out_type=[jax.ShapeDtypeStruct((2, _NT, ept), jnp.int32),
                  jax.ShapeDtypeStruct((2, _NT, ept), jnp.int32),
                  jax.ShapeDtypeStruct((2, _NT, 16), jnp.int32)],
        mesh=_mesh(),
        compiler_params=pltpu.CompilerParams(needs_layout_passes=False),
        scratch_types=[
            pltpu.VMEM((ept,), jnp.int32),
            pltpu.VMEM((ept,), jnp.int32),
            pltpu.VMEM((ept + 16,), jnp.int32),
            pltpu.VMEM((ept + 16,), jnp.int32),
            pltpu.VMEM((ept + 16,), jnp.int32),
            pltpu.VMEM((ept + 16,), jnp.int32),
            pltpu.VMEM((16,), jnp.int32),
        ],
    )
    def k(src_hbm, dst_hbm, srcL_hbm, dstL_hbm, cnt_hbm,
          siv, div, oAs, oAd, oBs, oBd, cv):
        co = lax.axis_index("c")
        s = lax.axis_index("s")
        blk = co * _NS + s
        pltpu.sync_copy(src_hbm.at[blk], siv)
        pltpu.sync_copy(dst_hbm.at[blk], div)

        zer = jnp.zeros((16,), jnp.int32)
        tra = jnp.full((16,), trash, jnp.int32)

        @pl.loop(0, ept + 16, step=16)
        def _pre(i):
            oAs[pl.ds(i, 16)] = zer
            oAd[pl.ds(i, 16)] = tra
            oBs[pl.ds(i, 16)] = zer
            oBd[pl.ds(i, 16)] = tra

        def body(j, carry):
            nA, nB = carry
            d = div[pl.ds(j * 16, 16)]
            sv = siv[pl.ds(j * 16, 16)]
            mA = d < half
            mB = jnp.logical_not(mA)
            plsc.store_compressed(oAs.at[pl.ds(nA, 16)], sv, mask=mA)
            plsc.store_compressed(oAd.at[pl.ds(nA, 16)], d, mask=mA)
            plsc.store_compressed(oBs.at[pl.ds(nB, 16)], sv, mask=mB)
            plsc.store_compressed(oBd.at[pl.ds(nB, 16)], d - half, mask=mB)
            nA = nA + jnp.max(plsc.all_reduce_population_count(mA))
            nB = nB + jnp.max(plsc.all_reduce_population_count(mB))
            return nA, nB

        nA, nB = lax.fori_loop(0, nv, body, (jnp.int32(0), jnp.int32(0)))

        i16 = lax.iota(jnp.int32, 16)
        cv[...] = jnp.where(i16 == 0, nA, 0)
        pltpu.sync_copy(cv, cnt_hbm.at[0, blk])
        cv[...] = jnp.where(i16 == 0, nB, 0)
        pltpu.sync_copy(cv, cnt_hbm.at[1, blk])
        pltpu.sync_copy(oAs.at[pl.ds(0, ept)], srcL_hbm.at[0, blk])
        pltpu.sync_copy(oAd.at[pl.ds(0, ept)], dstL_hbm.at[0, blk])
        pltpu.sync_copy(oBs.at[pl.ds(0, ept)], srcL_hbm.at[1, blk])
        pltpu.sync_copy(oBd.at[pl.ds(0, ept)], dstL_hbm.at[1, blk])

    return k(srcF, dstF)


def _sc_call(mode, hn, srcC, dstC, cnt, npad, ch):
    half = npad // 2
    accn = half + 640
    bw = half // _NS

    @functools.partial(
        pl.kernel,
        out_type=jax.ShapeDtypeStruct((npad, 128), jnp.float32),
        mesh=_mesh(),
        scratch_types=[
            pltpu.VMEM((ch, _C), jnp.int32),
            pltpu.VMEM((ch, _C), jnp.int32),
            pltpu.VMEM((_C, 128), jnp.float32),
            pltpu.VMEM((_C, 128), jnp.float32),
            pltpu.VMEM_SHARED((accn, 128), jnp.float32),
            pltpu.VMEM((16,), jnp.int32),
            pltpu.SemaphoreType.DMA,
            pltpu.SemaphoreType.DMA,
        ],
    )
    def k(hn_hbm, srcC_hbm, dstC_hbm, cnt_hbm, out_hbm,
          sl, dl, rows0, rows1, acc, cv, g0, g1):
        co = lax.axis_index("c")
        s = lax.axis_index("s")

        zrow = accn // _NS
        z16 = jnp.zeros((16,), jnp.float32)

        @pl.loop(0, _C)
        def _z(i):
            @pl.loop(0, 128, step=16)
            def _z2(q):
                rows0[i, pl.ds(q, 16)] = z16

        off = 0
        while off < zrow:
            n = min(_C, zrow - off)
            pltpu.sync_copy(rows0.at[pl.ds(0, n)],
                            acc.at[pl.ds(s * zrow + off, n)])
            off += n

        plsc.subcore_barrier()

        one = jnp.ones((16,), jnp.float32)
        if mode == "deg":
            @pl.loop(0, _C)
            def _ones(i):
                @pl.loop(0, 128, step=16)
                def _ones2(q):
                    rows0[i, pl.ds(q, 16)] = one

        for li in range(2):
            p = s + li * _NS
            pltpu.sync_copy(cnt_hbm.at[co, p], cv)
            pltpu.sync_copy(dstC_hbm.at[co, p], dl)
            c_p = cv[...][0]
            nch = (c_p + _C - 1) // _C

            if mode == "deg":
                def dbody(i, carry):
                    pltpu.sync_copy(rows0, acc.at[dl.at[i]], add=True)
                    return carry

                lax.fori_loop(0, nch, dbody, jnp.int32(0))
            else:
                pltpu.sync_copy(srcC_hbm.at[co, p], sl)
                nch = nch + lax.rem(nch, 2)

                @pl.when(nch > 0)
                def _prop():
                    pltpu.async_copy(hn_hbm.at[sl.at[0]], rows0, g0)
                    c1 = lax.rem(jnp.int32(1), nch)
                    pltpu.async_copy(hn_hbm.at[sl.at[c1]], rows1, g1)

                    def pbody(i, carry):
                        j = 2 * i
                        pltpu.make_async_copy(hn_hbm.at[sl.at[j]],
                                              rows0, g0).wait()
                        pltpu.sync_copy(rows0, acc.at[dl.at[j]], add=True)
                        jn = lax.rem(j + 2, nch)
                        pltpu.async_copy(hn_hbm.at[sl.at[jn]], rows0, g0)

                        j1 = j + 1
                        pltpu.make_async_copy(hn_hbm.at[sl.at[j1]],
                                              rows1, g1).wait()
                        pltpu.sync_copy(rows1, acc.at[dl.at[j1]], add=True)
                        jn1 = lax.rem(j + 3, nch)
                        pltpu.async_copy(hn_hbm.at[sl.at[jn1]], rows1, g1)
                        return carry

                    lax.fori_loop(0, nch // 2, pbody, jnp.int32(0))

                    pltpu.make_async_copy(hn_hbm.at[sl.at[0]],
                                          rows0, g0).wait()
                    pltpu.make_async_copy(hn_hbm.at[sl.at[0]],
                                          rows1, g1).wait()

        plsc.subcore_barrier()
        pltpu.sync_copy(acc.at[pl.ds(s * bw, bw)],
                        out_hbm.at[pl.ds(co * half + s * bw, bw)])

    return k(hn, srcC, dstC, cnt)


def _mm_body(x_ref, w_ref, b_ref, dp_ref, h_ref, hn_ref, dv_ref):
    h = jnp.dot(x_ref[...], w_ref[...],
                preferred_element_type=jnp.float32) + b_ref[...]
    deg = dp_ref[:, :1] + 1.0
    dv = jnp.broadcast_to(lax.rsqrt(deg), h.shape)
    h_ref[...] = h
    hn_ref[...] = h * dv
    dv_ref[...] = dv


def _mm_call(xp, w, b2, degp, npad):
    f = jax.ShapeDtypeStruct((npad, 128), jnp.float32)
    return pl.pallas_call(
        _mm_body,
        grid=(npad // _BM,),
        in_specs=[
            pl.BlockSpec((_BM, 128), lambda i: (i, 0)),
            pl.BlockSpec((128, 128), lambda i: (0, 0)),
            pl.BlockSpec((1, 128), lambda i: (0, 0)),
            pl.BlockSpec((_BM, 128), lambda i: (i, 0)),
        ],
        out_specs=[pl.BlockSpec((_BM, 128), lambda i: (i, 0))] * 3,
        out_shape=[f, f, f],
    )(xp, w, b2, degp)


def _comb_body(cl, dl, last, p_ref, hn_ref, h0_ref, dv_ref, h_ref, hno_ref):
    dv = dv_ref[...]
    agg = p_ref[...] + hn_ref[...]
    h = cl * (agg * dv) + dl * h0_ref[...]
    h_ref[...] = h
    if last:
        hno_ref[...] = h
    else:
        hno_ref[...] = h * dv


def _comb_call(p, hn, h0, dvb, cl, dl, last, npad):
    f = jax.ShapeDtypeStruct((npad, 128), jnp.float32)
    return pl.pallas_call(
        functools.partial(_comb_body, cl, dl, last),
        grid=(npad // _BM,),
        in_specs=[pl.BlockSpec((_BM, 128), lambda i: (i, 0))] * 4,
        out_specs=[pl.BlockSpec((_BM, 128), lambda i: (i, 0))] * 2,
        out_shape=[f, f],
    )(p, hn, h0, dvb)


def kernel(features, edge_index, W, b):
    n, d = features.shape
    e = edge_index.shape[1]
    npad = (n // 2560 + 1) * 2560

    ept0 = -(-e // _NT)
    ch = -(-ept0 // _C)
    ch += ch % 2
    ept = ch * _C

    srcF = jnp.pad(edge_index[0], (0, _NT * ept - e)).reshape(_NT, ept)
    dstF = jnp.pad(edge_index[1], (0, _NT * ept - e),
                   constant_values=npad).reshape(_NT, ept)
    xp = jnp.pad(features, ((0, npad - n), (0, 0)))
    b2 = b.reshape(1, d)

    srcL, dstL, cnt = _part_call(srcF, dstF, npad, ch)
    srcC = srcL.reshape(2, _NT, ept // _C, _C)
    dstC = dstL.reshape(2, _NT, ept // _C, _C)

    degp = _sc_call("deg", xp, srcC, dstC, cnt, npad, ch)
    h0, hn, dvb = _mm_call(xp, W, b2, degp, npad)

    h = h0
    for l in range(1, _K + 1):
        p = _sc_call("prop", hn, srcC, dstC, cnt, npad, ch)
        h, hn = _comb_call(p, hn, h0, dvb,
                           l / (l + 1.0), 1.0 / (l + 1.0), l == _K, npad)
    return h[:n]
